# Initial kernel scaffold; baseline (speedup 1.0000x reference)
import functools

import jax
import jax.numpy as jnp
from jax import lax
from jax.experimental import pallas as pl
from jax.experimental.pallas import tpu as pltpu
from jax.experimental.pallas import tpu_sc as plsc

N_NODES = 10000
N_EDGES = 320000
D = 128

NCORE = 2
NSUB = 16
NW = NCORE * NSUB
EPW = N_EDGES // NW
CHUNK = 80
NCHUNK = EPW // CHUNK
ROWS_PER_SUB = N_NODES // NSUB


def _sc_body(feat_hbm, src_hbm, dst_hbm, ew_hbm, out_hbm,
             rows_v, src_v, dst_v, w_v, acc_sh, sem):
    c = lax.axis_index("c")
    s = lax.axis_index("s")
    wid = c * NSUB + s

    def zrow(i, carry):
        for j in range(8):
            rows_v[i, pl.ds(j * 16, 16)] = jnp.zeros((16,), jnp.float32)
        return carry

    lax.fori_loop(0, CHUNK, zrow, 0)

    base = s * ROWS_PER_SUB
    nfull = ROWS_PER_SUB // CHUNK
    rem = ROWS_PER_SUB - nfull * CHUNK
    for k in range(nfull):
        pltpu.sync_copy(rows_v, acc_sh.at[pl.ds(base + k * CHUNK, CHUNK)])
    if rem:
        pltpu.sync_copy(rows_v.at[pl.ds(0, rem)],
                        acc_sh.at[pl.ds(base + nfull * CHUNK, rem)])
    plsc.subcore_barrier()

    ebase = wid * EPW

    def chunk_body(k, carry):
        off = ebase + k * CHUNK
        pltpu.sync_copy(src_hbm.at[pl.ds(off, CHUNK)], src_v)
        pltpu.sync_copy(dst_hbm.at[pl.ds(off, CHUNK)], dst_v)
        pltpu.sync_copy(ew_hbm.at[pl.ds(off, CHUNK)], w_v)
        pltpu.async_copy(feat_hbm.at[src_v], rows_v, sem).wait()

        def edge_body(e, ecarry):
            wvec = plsc.load_gather(w_v, [jnp.full((16,), e, jnp.int32)])
            for j in range(8):
                sl = pl.ds(j * 16, 16)
                rows_v[e, sl] = rows_v[e, sl] * wvec
            return ecarry

        lax.fori_loop(0, CHUNK, edge_body, 0)
        pltpu.sync_copy(rows_v, acc_sh.at[dst_v], add=True)
        return carry

    lax.fori_loop(0, NCHUNK, chunk_body, 0)
    plsc.subcore_barrier()

    pltpu.sync_copy(acc_sh.at[pl.ds(base, ROWS_PER_SUB)],
                    out_hbm.at[c, pl.ds(base, ROWS_PER_SUB)])


_sc_aggregate = pl.kernel(
    _sc_body,
    out_type=jax.ShapeDtypeStruct((NCORE, N_NODES, D), jnp.float32),
    mesh=plsc.VectorSubcoreMesh(core_axis_name="c", subcore_axis_name="s"),
    scratch_types=[
        pltpu.VMEM((CHUNK, D), jnp.float32),
        pltpu.VMEM((CHUNK,), jnp.int32),
        pltpu.VMEM((CHUNK,), jnp.int32),
        pltpu.VMEM((CHUNK,), jnp.float32),
        pltpu.VMEM_SHARED((N_NODES, D), jnp.float32),
        pltpu.SemaphoreType.DMA,
    ],
)

ROW_BLK = 1000


def _tc_body(p_ref, w_ref, o_ref):
    acc = p_ref[0] + p_ref[1]
    o_ref[...] = jnp.maximum(
        jnp.dot(acc, w_ref[...], preferred_element_type=jnp.float32), 0.0)


def _tc_finish(partials, W):
    return pl.pallas_call(
        _tc_body,
        grid=(N_NODES // ROW_BLK,),
        in_specs=[
            pl.BlockSpec((NCORE, ROW_BLK, D), lambda i: (0, i, 0)),
            pl.BlockSpec((D, D), lambda i: (0, 0)),
        ],
        out_specs=pl.BlockSpec((ROW_BLK, D), lambda i: (i, 0)),
        out_shape=jax.ShapeDtypeStruct((N_NODES, D), jnp.float32),
    )(partials, W)


@jax.jit
def kernel(feat, edge_index, edge_weight, W):
    src = edge_index[0]
    dst = edge_index[1]
    partials = _sc_aggregate(feat, src, dst, edge_weight)
    return _tc_finish(partials, W)


# speedup vs baseline: 4.5560x; 4.5560x over previous
import functools

import jax
import jax.numpy as jnp
from jax import lax
from jax.experimental import pallas as pl
from jax.experimental.pallas import tpu as pltpu
from jax.experimental.pallas import tpu_sc as plsc

N_NODES = 10000
N_EDGES = 320000
D = 128

NCORE = 2
NSUB = 16
NW = NCORE * NSUB
EPW = N_EDGES // NW
CHUNK = 80
NCHUNK = EPW // CHUNK
ROWS_PER_SUB = 624
TAIL_ROWS = N_NODES - NSUB * ROWS_PER_SUB


def _sc_body(feat_hbm, src_hbm, dst_hbm, ew_hbm, out_hbm,
             rows_v, src_v, dst_v, w_v, acc_sh, sem):
    c = lax.axis_index("c")
    s = lax.axis_index("s")
    wid = c * NSUB + s

    def zrow(i, carry):
        for j in range(8):
            rows_v[i, pl.ds(j * 16, 16)] = jnp.zeros((16,), jnp.float32)
        return carry

    lax.fori_loop(0, CHUNK, zrow, 0)

    base = s * ROWS_PER_SUB
    nfull = ROWS_PER_SUB // CHUNK
    rem = ROWS_PER_SUB - nfull * CHUNK
    for k in range(nfull):
        pltpu.sync_copy(rows_v, acc_sh.at[pl.ds(base + k * CHUNK, CHUNK)])
    if rem:
        pltpu.sync_copy(rows_v.at[pl.ds(0, rem)],
                        acc_sh.at[pl.ds(base + nfull * CHUNK, rem)])

    @pl.when(s == NSUB - 1)
    def _zero_tail():
        pltpu.sync_copy(rows_v.at[pl.ds(0, TAIL_ROWS)],
                        acc_sh.at[pl.ds(NSUB * ROWS_PER_SUB, TAIL_ROWS)])

    plsc.subcore_barrier()

    ebase = wid * EPW

    def chunk_body(k, carry):
        off = ebase + k * CHUNK
        pltpu.sync_copy(src_hbm.at[pl.ds(off, CHUNK)], src_v)
        pltpu.sync_copy(dst_hbm.at[pl.ds(off, CHUNK)], dst_v)
        pltpu.sync_copy(ew_hbm.at[pl.ds(off, CHUNK)], w_v)
        pltpu.async_copy(feat_hbm.at[src_v], rows_v, sem).wait()

        def group_body(g, gcarry):
            w16 = w_v[pl.ds(g * 16, 16)]
            for l in range(16):
                wvec = jnp.full((16,), w16[l], jnp.float32)
                e = g * 16 + l
                for j in range(8):
                    sl = pl.ds(j * 16, 16)
                    rows_v[e, sl] = rows_v[e, sl] * wvec
            return gcarry

        lax.fori_loop(0, CHUNK // 16, group_body, 0)
        pltpu.sync_copy(rows_v, acc_sh.at[dst_v], add=True)
        return carry

    lax.fori_loop(0, NCHUNK, chunk_body, 0)
    plsc.subcore_barrier()

    pltpu.sync_copy(acc_sh.at[pl.ds(base, ROWS_PER_SUB)],
                    out_hbm.at[c, pl.ds(base, ROWS_PER_SUB)])

    @pl.when(s == NSUB - 1)
    def _flush_tail():
        pltpu.sync_copy(acc_sh.at[pl.ds(NSUB * ROWS_PER_SUB, TAIL_ROWS)],
                        out_hbm.at[c, pl.ds(NSUB * ROWS_PER_SUB, TAIL_ROWS)])


_sc_aggregate = pl.kernel(
    _sc_body,
    out_type=jax.ShapeDtypeStruct((NCORE, N_NODES, D), jnp.float32),
    mesh=plsc.VectorSubcoreMesh(core_axis_name="c", subcore_axis_name="s"),
    scratch_types=[
        pltpu.VMEM((CHUNK, D), jnp.float32),
        pltpu.VMEM((CHUNK,), jnp.int32),
        pltpu.VMEM((CHUNK,), jnp.int32),
        pltpu.VMEM((CHUNK,), jnp.float32),
        pltpu.VMEM_SHARED((N_NODES, D), jnp.float32),
        pltpu.SemaphoreType.DMA,
    ],
)

ROW_BLK = 1000


def _tc_body(p_ref, w_ref, o_ref):
    acc = p_ref[0] + p_ref[1]
    o_ref[...] = jnp.maximum(
        jnp.dot(acc, w_ref[...], preferred_element_type=jnp.float32), 0.0)


def _tc_finish(partials, W):
    return pl.pallas_call(
        _tc_body,
        grid=(N_NODES // ROW_BLK,),
        in_specs=[
            pl.BlockSpec((NCORE, ROW_BLK, D), lambda i: (0, i, 0)),
            pl.BlockSpec((D, D), lambda i: (0, 0)),
        ],
        out_specs=pl.BlockSpec((ROW_BLK, D), lambda i: (i, 0)),
        out_shape=jax.ShapeDtypeStruct((N_NODES, D), jnp.float32),
    )(partials, W)


@jax.jit
def kernel(feat, edge_index, edge_weight, W):
    src = edge_index[0]
    dst = edge_index[1]
    partials = _sc_aggregate(feat, src, dst, edge_weight)
    return _tc_finish(partials, W)
